# baseline (speedup 1.0000x reference)
import functools

import jax
import jax.numpy as jnp
from jax import lax
from jax.experimental import pallas as pl
from jax.experimental.pallas import tpu as pltpu
from jax.experimental.pallas import tpu_sc as plsc

_ROWS = 8192
_DIM = 1024
_NC = 2
_NS = 16
_NW = _NC * _NS
_ROWS_PER_W = _ROWS // _NW
_CH = 32
_NB = 3
_NCHUNK = _ROWS_PER_W // _CH

_mesh = plsc.VectorSubcoreMesh(core_axis_name="c", subcore_axis_name="s")


@functools.partial(
    pl.kernel,
    mesh=_mesh,
    out_type=jax.ShapeDtypeStruct((_ROWS, _DIM), jnp.float32),
    scratch_types=[
        pltpu.VMEM((_NB, _CH, _DIM), jnp.float32),
        pltpu.SemaphoreType.DMA((_NB,)),
        pltpu.SemaphoreType.DMA((_NB,)),
    ],
)
def _sc_copy(src_hbm, out_hbm, buf, in_sems, out_sems):
    wid = lax.axis_index("s") * _NC + lax.axis_index("c")
    base = wid * _ROWS_PER_W

    def in_cp(i, b):
        return pltpu.make_async_copy(
            src_hbm.at[pl.ds(base + i * _CH, _CH), :], buf.at[b], in_sems.at[b])

    def out_cp(i, b):
        return pltpu.make_async_copy(
            buf.at[b], out_hbm.at[pl.ds(base + i * _CH, _CH), :], out_sems.at[b])

    for i in range(min(_NB, _NCHUNK)):
        in_cp(i, i).start()
    for i in range(_NCHUNK):
        b = i % _NB
        in_cp(i, b).wait()
        out_cp(i, b).start()
        nxt = i + _NB
        if nxt < _NCHUNK:
            out_cp(i, b).wait()
            in_cp(nxt, b).start()
    for i in range(max(0, _NCHUNK - _NB), _NCHUNK):
        out_cp(i, i % _NB).wait()


def kernel(input_ids, pos_table):
    del input_ids
    return _sc_copy(pos_table)


# speedup vs baseline: 1.0061x; 1.0061x over previous
import functools

import jax
import jax.numpy as jnp
from jax import lax
from jax.experimental import pallas as pl
from jax.experimental.pallas import tpu as pltpu
from jax.experimental.pallas import tpu_sc as plsc

_ROWS = 8192
_DIM = 1024
_NC = 2
_NS = 16
_NW = _NC * _NS
_ROWS_PER_W = _ROWS // _NW
_CH = 16
_NB = 7
_LOOKAHEAD = 5
_NCHUNK = _ROWS_PER_W // _CH

_mesh = plsc.VectorSubcoreMesh(core_axis_name="c", subcore_axis_name="s")


@functools.partial(
    pl.kernel,
    mesh=_mesh,
    out_type=jax.ShapeDtypeStruct((_ROWS, _DIM), jnp.float32),
    scratch_types=[
        pltpu.VMEM((_NB, _CH, _DIM), jnp.float32),
        pltpu.SemaphoreType.DMA((_NB,)),
        pltpu.SemaphoreType.DMA((_NB,)),
    ],
)
def _sc_copy(src_hbm, out_hbm, buf, in_sems, out_sems):
    wid = lax.axis_index("s") * _NC + lax.axis_index("c")
    base = wid * _ROWS_PER_W

    def in_cp(i, b):
        return pltpu.make_async_copy(
            src_hbm.at[pl.ds(base + i * _CH, _CH), :], buf.at[b], in_sems.at[b])

    def out_cp(i, b):
        return pltpu.make_async_copy(
            buf.at[b], out_hbm.at[pl.ds(base + i * _CH, _CH), :], out_sems.at[b])

    for i in range(min(_LOOKAHEAD, _NCHUNK)):
        in_cp(i, i % _NB).start()
    for i in range(_NCHUNK):
        b = i % _NB
        in_cp(i, b).wait()
        out_cp(i, b).start()
        nxt = i + _LOOKAHEAD
        if nxt < _NCHUNK:
            bn = nxt % _NB
            prev = nxt - _NB
            if prev >= 0:
                out_cp(prev, bn).wait()
            in_cp(nxt, bn).start()
    for i in range(max(0, _NCHUNK - _NB), _NCHUNK):
        out_cp(i, i % _NB).wait()


def kernel(input_ids, pos_table):
    del input_ids
    return _sc_copy(pos_table)


# speedup vs baseline: 1.9920x; 1.9800x over previous
import jax
import jax.numpy as jnp
from jax.experimental import pallas as pl
from jax.experimental.pallas import tpu as pltpu

_BLOCK_ROWS = 2048


def _copy_body(src_ref, out_ref):
    out_ref[...] = src_ref[...]


def kernel(input_ids, pos_table):
    seq_len = input_ids.shape[1]
    rows, dim = pos_table.shape
    assert seq_len == rows
    grid = (rows // _BLOCK_ROWS,)
    return pl.pallas_call(
        _copy_body,
        grid=grid,
        in_specs=[pl.BlockSpec((_BLOCK_ROWS, dim), lambda i: (i, 0))],
        out_specs=pl.BlockSpec((_BLOCK_ROWS, dim), lambda i: (i, 0)),
        out_shape=jax.ShapeDtypeStruct((seq_len, dim), pos_table.dtype),
        compiler_params=pltpu.CompilerParams(
            dimension_semantics=("parallel",),
        ),
    )(pos_table)
